# baseline (speedup 1.0000x reference)
import functools

import jax
import jax.numpy as jnp
from jax import lax
from jax.experimental import pallas as pl
from jax.experimental.pallas import tpu as pltpu
from jax.experimental.pallas import tpu_sc as plsc

K = 16
B = 4
N = 4096

SC_N = 896
TC_N = N - SC_N
R = 1600
NSLICE = 32
S = N // NSLICE

NW = 32
WPB = NW // B
RPW = SC_N // WPB
L = 16
NCELL = 8
JCH = (N // L) // NCELL


def _knn_block(pts_all_ref, pts_rows_ref, out_ref):
    pts_all = pts_all_ref[0]
    pts_rows = pts_rows_ref[0]

    inf = jnp.float32(jnp.inf)

    sqx = jnp.sum(pts_rows * pts_rows, axis=1, keepdims=True)
    sqy = jnp.sum(pts_all * pts_all, axis=0, keepdims=True)
    gramn = jax.lax.dot_general(
        pts_rows * -2.0, pts_all, (((1,), (0,)), ((), ())),
        preferred_element_type=jnp.float32,
    )

    m1 = jnp.full((R, S), inf, dtype=jnp.float32)
    m2 = jnp.full((R, S), inf, dtype=jnp.float32)
    for j in range(NSLICE):
        sl = slice(j * S, (j + 1) * S)
        d2 = (sqx + sqy[:, sl]) + gramn[:, sl]
        m2 = jnp.minimum(m2, jnp.maximum(m1, d2))
        m1 = jnp.minimum(m1, d2)

    cand = jnp.concatenate([m1, m2], axis=1)

    acc = jnp.zeros((1, 1), dtype=jnp.float32)
    mprev = jnp.full((R, 1), -jnp.inf, dtype=jnp.float32)
    for t in range(K + 1):
        masked = jnp.where(cand > mprev, cand, inf)
        m = jnp.min(masked, axis=1, keepdims=True)
        if t > 0:
            mc = jnp.maximum(m, 0.0)
            acc = acc + jnp.sum(jnp.sqrt(mc + 1e-12)).reshape(1, 1)
        mprev = m

    out_ref[0, 0] = jnp.broadcast_to(acc, (8, 128))


def _gather16(x, idx):
    dnums = lax.GatherDimensionNumbers(
        offset_dims=(), collapsed_slice_dims=(0,), start_index_map=(0,))
    return lax.gather(x, idx[:, None], dnums, (1,),
                      mode=lax.GatherScatterMode.PROMISE_IN_BOUNDS)


def _sqrt16(x):
    bits = lax.bitcast_convert_type(x, jnp.int32)
    y = lax.bitcast_convert_type(jnp.int32(0x5F3759DF) - (bits >> 1),
                                 jnp.float32)
    for _ in range(3):
        y = y * (1.5 - 0.5 * x * y * y)
    return x * y


_sc_mesh = plsc.VectorSubcoreMesh(core_axis_name="c", subcore_axis_name="s")


@functools.partial(
    pl.kernel,
    mesh=_sc_mesh,
    out_type=jax.ShapeDtypeStruct((NW, L), jnp.float32),
    scratch_types=[
        pltpu.VMEM((3, N), jnp.float32),
        pltpu.VMEM((3, RPW, L), jnp.float32),
        pltpu.VMEM((L,), jnp.float32),
    ],
)
def _sc_knn(pts_hbm, rs_hbm, out_hbm, pts_v, rs_v, acc_v):
    wid = lax.axis_index("s") * 2 + lax.axis_index("c")
    b = wid // WPB
    kk = wid % WPB

    pltpu.sync_copy(pts_hbm.at[b], pts_v)
    pltpu.sync_copy(rs_hbm.at[b, kk], rs_v)

    inf = jnp.float32(jnp.inf)
    iota = lax.iota(jnp.int32, L)
    rot_idx = [(iota + sh) & (L - 1) for sh in (8, 4, 2, 1)]

    def row_body(i, vacc):
        xs = rs_v[0, i]
        ys = rs_v[1, i]
        zs = rs_v[2, i]

        def fold_body(jo, carry):
            ms = list(carry)
            for ks in range(NCELL):
                j = jo * NCELL + ks
                px = pts_v[0, pl.ds(j * L, L)]
                py = pts_v[1, pl.ds(j * L, L)]
                pz = pts_v[2, pl.ds(j * L, L)]
                dx = xs - px
                dy = ys - py
                dz = zs - pz
                d2 = dx * dx + dy * dy + dz * dz
                ms[NCELL + ks] = jnp.minimum(
                    ms[NCELL + ks], jnp.maximum(ms[ks], d2))
                ms[ks] = jnp.minimum(ms[ks], d2)
            return tuple(ms)

        init = tuple(jnp.full((L,), inf, jnp.float32) for _ in range(2 * NCELL))
        cands = lax.fori_loop(0, JCH, fold_body, init)

        sel = jnp.zeros((L,), jnp.float32)
        mprev = jnp.full((L,), -jnp.inf, jnp.float32)
        for t in range(K + 1):
            mv = jnp.full((L,), inf, jnp.float32)
            for v in cands:
                mv = jnp.minimum(mv, jnp.where(v > mprev, v, inf))
            for ridx in rot_idx:
                mv = jnp.minimum(mv, _gather16(mv, ridx))
            if t > 0:
                sel = jnp.where(iota == (t - 1), mv, sel)
            mprev = mv

        return vacc + _sqrt16(sel + 1e-12)

    vacc = lax.fori_loop(0, RPW, row_body, jnp.zeros((L,), jnp.float32))
    acc_v[...] = vacc
    pltpu.sync_copy(acc_v, out_hbm.at[wid])


@jax.jit
def kernel(pcs):
    pcs_t = jnp.swapaxes(pcs, 1, 2)

    tc_parts = pl.pallas_call(
        _knn_block,
        grid=(B, TC_N // R),
        in_specs=[
            pl.BlockSpec((1, 3, N), lambda b, r: (b, 0, 0)),
            pl.BlockSpec((1, R, 3), lambda b, r: (b, r, 0)),
        ],
        out_specs=pl.BlockSpec((1, 1, 8, 128), lambda b, r: (b, r, 0, 0)),
        out_shape=jax.ShapeDtypeStruct((B, TC_N // R, 8, 128), jnp.float32),
        compiler_params=pltpu.CompilerParams(
            dimension_semantics=("parallel", "parallel"),
        ),
    )(pcs_t, pcs)

    rs = jnp.broadcast_to(
        pcs_t[:, :, TC_N:].reshape(B, 3, WPB, RPW, 1),
        (B, 3, WPB, RPW, L),
    ).transpose(0, 2, 1, 3, 4)
    sc_parts = _sc_knn(pcs_t, rs)

    total = jnp.sum(tc_parts[:, :, 0, 0]) + jnp.sum(sc_parts)
    return total / jnp.float32(B * N * K)


# speedup vs baseline: 1.1044x; 1.1044x over previous
import functools

import jax
import jax.numpy as jnp
from jax import lax
from jax.experimental import pallas as pl
from jax.experimental.pallas import tpu as pltpu
from jax.experimental.pallas import tpu_sc as plsc

K = 16
B = 4
N = 4096

SC_N = 768
TC_N = N - SC_N
R = 1664
NSLICE = 32
S = N // NSLICE

NW = 32
WPB = NW // B
RPW = SC_N // WPB
L = 16
NCELL = 8
JCH = (N // L) // NCELL


def _knn_block(pts_all_ref, pts_rows_ref, out_ref):
    pts_all = pts_all_ref[0]
    pts_rows = pts_rows_ref[0]

    inf = jnp.float32(jnp.inf)

    sqx = jnp.sum(pts_rows * pts_rows, axis=1, keepdims=True)
    sqy = jnp.sum(pts_all * pts_all, axis=0, keepdims=True)
    gramn = jax.lax.dot_general(
        pts_rows * -2.0, pts_all, (((1,), (0,)), ((), ())),
        preferred_element_type=jnp.float32,
    )

    m1 = jnp.full((R, S), inf, dtype=jnp.float32)
    m2 = jnp.full((R, S), inf, dtype=jnp.float32)
    for j in range(NSLICE):
        sl = slice(j * S, (j + 1) * S)
        d2 = (sqx + sqy[:, sl]) + gramn[:, sl]
        m2 = jnp.minimum(m2, jnp.maximum(m1, d2))
        m1 = jnp.minimum(m1, d2)

    cand = jnp.concatenate([m1, m2], axis=1)

    acc = jnp.zeros((1, 1), dtype=jnp.float32)
    mprev = jnp.full((R, 1), -jnp.inf, dtype=jnp.float32)
    for t in range(K + 1):
        masked = jnp.where(cand > mprev, cand, inf)
        m = jnp.min(masked, axis=1, keepdims=True)
        if t > 0:
            mc = jnp.maximum(m, 0.0)
            acc = acc + jnp.sum(jnp.sqrt(mc + 1e-12)).reshape(1, 1)
        mprev = m

    out_ref[0, 0] = jnp.broadcast_to(acc, (8, 128))


def _gather16(x, idx):
    dnums = lax.GatherDimensionNumbers(
        offset_dims=(), collapsed_slice_dims=(0,), start_index_map=(0,))
    return lax.gather(x, idx[:, None], dnums, (1,),
                      mode=lax.GatherScatterMode.PROMISE_IN_BOUNDS)


def _sqrt16(x):
    bits = lax.bitcast_convert_type(x, jnp.int32)
    y = lax.bitcast_convert_type(jnp.int32(0x5F3759DF) - (bits >> 1),
                                 jnp.float32)
    for _ in range(3):
        y = y * (1.5 - 0.5 * x * y * y)
    return x * y


_sc_mesh = plsc.VectorSubcoreMesh(core_axis_name="c", subcore_axis_name="s")


@functools.partial(
    pl.kernel,
    mesh=_sc_mesh,
    out_type=jax.ShapeDtypeStruct((NW, L), jnp.float32),
    scratch_types=[
        pltpu.VMEM((3, N), jnp.float32),
        pltpu.VMEM((3, RPW, L), jnp.float32),
        pltpu.VMEM((L,), jnp.float32),
    ],
)
def _sc_knn(pts_hbm, rs_hbm, out_hbm, pts_v, rs_v, acc_v):
    wid = lax.axis_index("s") * 2 + lax.axis_index("c")
    b = wid // WPB
    kk = wid % WPB

    pltpu.sync_copy(pts_hbm.at[b], pts_v)
    pltpu.sync_copy(rs_hbm.at[b, kk], rs_v)

    inf = jnp.float32(jnp.inf)
    iota = lax.iota(jnp.int32, L)
    rot_idx = [(iota + sh) & (L - 1) for sh in (8, 4, 2, 1)]

    def row_body(i, vacc):
        xs = rs_v[0, i]
        ys = rs_v[1, i]
        zs = rs_v[2, i]

        def fold_body(jo, carry):
            ms = list(carry)
            for ks in range(NCELL):
                j = jo * NCELL + ks
                px = pts_v[0, pl.ds(j * L, L)]
                py = pts_v[1, pl.ds(j * L, L)]
                pz = pts_v[2, pl.ds(j * L, L)]
                dx = xs - px
                dy = ys - py
                dz = zs - pz
                d2 = dx * dx + dy * dy + dz * dz
                ms[NCELL + ks] = jnp.minimum(
                    ms[NCELL + ks], jnp.maximum(ms[ks], d2))
                ms[ks] = jnp.minimum(ms[ks], d2)
            return tuple(ms)

        init = tuple(jnp.full((L,), inf, jnp.float32) for _ in range(2 * NCELL))
        cands = lax.fori_loop(0, JCH, fold_body, init)

        sel = jnp.zeros((L,), jnp.float32)
        mprev = jnp.full((L,), -jnp.inf, jnp.float32)
        for t in range(K + 1):
            mv = jnp.full((L,), inf, jnp.float32)
            for v in cands:
                mv = jnp.minimum(mv, jnp.where(v > mprev, v, inf))
            for ridx in rot_idx:
                mv = jnp.minimum(mv, _gather16(mv, ridx))
            if t > 0:
                sel = jnp.where(iota == (t - 1), mv, sel)
            mprev = mv

        return vacc + _sqrt16(sel + 1e-12)

    vacc = lax.fori_loop(0, RPW, row_body, jnp.zeros((L,), jnp.float32))
    acc_v[...] = vacc
    pltpu.sync_copy(acc_v, out_hbm.at[wid])


@jax.jit
def kernel(pcs):
    pcs_t = jnp.swapaxes(pcs, 1, 2)

    tc_parts = pl.pallas_call(
        _knn_block,
        grid=(B, TC_N // R),
        in_specs=[
            pl.BlockSpec((1, 3, N), lambda b, r: (b, 0, 0)),
            pl.BlockSpec((1, R, 3), lambda b, r: (b, r, 0)),
        ],
        out_specs=pl.BlockSpec((1, 1, 8, 128), lambda b, r: (b, r, 0, 0)),
        out_shape=jax.ShapeDtypeStruct((B, TC_N // R, 8, 128), jnp.float32),
        compiler_params=pltpu.CompilerParams(
            dimension_semantics=("parallel", "parallel"),
        ),
    )(pcs_t, pcs)

    rs = jnp.broadcast_to(
        pcs_t[:, :, TC_N:].reshape(B, 3, WPB, RPW, 1),
        (B, 3, WPB, RPW, L),
    ).transpose(0, 2, 1, 3, 4)
    sc_parts = _sc_knn(pcs_t, rs)

    total = jnp.sum(tc_parts[:, :, 0, 0]) + jnp.sum(sc_parts)
    return total / jnp.float32(B * N * K)
